# Initial kernel scaffold; baseline (speedup 1.0000x reference)
import jax
import jax.numpy as jnp
from jax.experimental import pallas as pl


def kernel(edge_index, node_feats, edge_feats, eps, W1, b1, gamma1, beta1, W2, b2, gamma2, beta2):
    raise NotImplementedError("write your pallas kernel here")


# speedup vs baseline: 3.2897x; 3.2897x over previous
import functools

import jax
import jax.numpy as jnp
from jax import lax
from jax.experimental import pallas as pl
from jax.experimental.pallas import tpu as pltpu
from jax.experimental.pallas import tpu_sc as plsc

_N = 10000
_E = 320000
_D = 128
_BN_EPS = 1e-5

_NC = 2
_NS = 16
_NW = _NC * _NS
_EPW = _E // _NW
_C = 80
_NCH = _EPW // _C
_NP = 10240
_RPS = _NP // _NS


def _sc_body(src_h, dst_h, node_h, edge_h, zf_h,
             psum_h,
             srcv, dstv, rows, erows, acc, sem):
    c = lax.axis_index("c")
    s = lax.axis_index("s")
    wid = s * _NC + c
    r0 = s * _RPS
    nb = _RPS // _C

    pltpu.sync_copy(zf_h, rows)
    for i in range(nb):
        pltpu.sync_copy(rows, acc.at[pl.ds(r0 + i * _C, _C)])
    plsc.subcore_barrier()

    def chunk(j, carry):
        base = wid * _EPW + j * _C
        pltpu.sync_copy(src_h.at[pl.ds(base, _C)], srcv)
        pltpu.sync_copy(dst_h.at[pl.ds(base, _C)], dstv)
        pltpu.sync_copy(edge_h.at[pl.ds(base, _C)], erows)
        pltpu.async_copy(node_h.at[srcv], rows, sem).wait()
        pltpu.sync_copy(rows, acc.at[dstv], add=True)
        pltpu.sync_copy(erows, acc.at[dstv], add=True)
        return carry

    lax.fori_loop(0, _NCH, chunk, 0)

    plsc.subcore_barrier()
    for i in range(nb):
        pltpu.sync_copy(acc.at[pl.ds(r0 + i * _C, _C)], rows)
        pltpu.sync_copy(rows, psum_h.at[c, pl.ds(r0 + i * _C, _C)])


def _sc_count_body(dst_h, ones_h, zf_h, pcnt_h,
                   dstv, onesb, rows, acc, sem):
    c = lax.axis_index("c")
    s = lax.axis_index("s")
    wid = s * _NC + c
    r0 = s * _RPS
    nb = _RPS // _C

    pltpu.sync_copy(zf_h, rows)
    for i in range(nb):
        pltpu.sync_copy(rows, acc.at[pl.ds(r0 + i * _C, _C)])
    pltpu.sync_copy(ones_h, onesb)
    plsc.subcore_barrier()

    def chunk(j, carry):
        base = wid * _EPW + j * _C
        pltpu.sync_copy(dst_h.at[pl.ds(base, _C)], dstv)
        pltpu.sync_copy(onesb, acc.at[dstv], add=True)
        return carry

    lax.fori_loop(0, _NCH, chunk, 0)

    plsc.subcore_barrier()
    for i in range(nb):
        pltpu.sync_copy(acc.at[pl.ds(r0 + i * _C, _C)], rows)
        pltpu.sync_copy(rows, pcnt_h.at[c, pl.ds(r0 + i * _C, _C)])


def _sc_scatter(src, dst, node_feats, edge_feats):
    zf = jnp.zeros((_C, _D), jnp.float32)
    ones_blk = jnp.ones((_C, _D), jnp.float32)
    mesh = plsc.VectorSubcoreMesh(core_axis_name="c", subcore_axis_name="s",
                                  num_cores=_NC, num_subcores=_NS)
    sum_call = pl.kernel(
        _sc_body,
        out_type=jax.ShapeDtypeStruct((_NC, _NP, _D), jnp.float32),
        mesh=mesh,
        scratch_types=[
            pltpu.VMEM((_C,), jnp.int32),
            pltpu.VMEM((_C,), jnp.int32),
            pltpu.VMEM((_C, _D), jnp.float32),
            pltpu.VMEM((_C, _D), jnp.float32),
            pltpu.VMEM_SHARED((_NP, _D), jnp.float32),
            pltpu.SemaphoreType.DMA,
        ],
    )
    cnt_call = pl.kernel(
        _sc_count_body,
        out_type=jax.ShapeDtypeStruct((_NC, _NP, _D), jnp.float32),
        mesh=mesh,
        scratch_types=[
            pltpu.VMEM((_C,), jnp.int32),
            pltpu.VMEM((_C, _D), jnp.float32),
            pltpu.VMEM((_C, _D), jnp.float32),
            pltpu.VMEM_SHARED((_NP, _D), jnp.float32),
            pltpu.SemaphoreType.DMA,
        ],
    )
    psum = sum_call(src, dst, node_feats, edge_feats, zf)
    pcnt = cnt_call(dst, ones_blk, zf)
    return psum, pcnt


def _tc_body(x_r, ps_r, pc_r, eps_r, w1_r, b1_r, g1_r, be1_r,
             w2_r, b2_r, g2_r, be2_r, o_r):
    summed = ps_r[0, :_N] + ps_r[1, :_N]
    cnt = pc_r[0, :_N, 0:1] + pc_r[1, :_N, 0:1]
    hv = summed / jnp.maximum(cnt, 1.0)
    h = (1.0 + eps_r[0, 0]) * x_r[...] + hv
    y = lax.dot_general(h, w1_r[...], (((1,), (1,)), ((), ())),
                        preferred_element_type=jnp.float32,
                        precision=lax.Precision.HIGHEST)
    y = y + b1_r[...]
    mu = jnp.mean(y, axis=0, keepdims=True)
    d = y - mu
    var = jnp.mean(d * d, axis=0, keepdims=True)
    y = g1_r[...] * (d * lax.rsqrt(var + _BN_EPS)) + be1_r[...]
    y = jnp.maximum(y, 0.0)
    z = lax.dot_general(y, w2_r[...], (((1,), (1,)), ((), ())),
                        preferred_element_type=jnp.float32,
                        precision=lax.Precision.HIGHEST)
    z = z + b2_r[...]
    mu2 = jnp.mean(z, axis=0, keepdims=True)
    d2 = z - mu2
    var2 = jnp.mean(d2 * d2, axis=0, keepdims=True)
    z = g2_r[...] * (d2 * lax.rsqrt(var2 + _BN_EPS)) + be2_r[...]
    o_r[...] = jnp.maximum(z, 0.0)


def _tc_mlp(node_feats, psum, pcnt, eps, W1, b1, g1, be1, W2, b2, g2, be2):
    return pl.pallas_call(
        _tc_body,
        out_shape=jax.ShapeDtypeStruct((_N, _D), jnp.float32),
        compiler_params=pltpu.CompilerParams(
            vmem_limit_bytes=128 * 1024 * 1024),
    )(node_feats, psum, pcnt,
      eps.reshape(1, 1),
      W1, b1.reshape(1, 2 * _D), g1.reshape(1, 2 * _D), be1.reshape(1, 2 * _D),
      W2, b2.reshape(1, _D), g2.reshape(1, _D), be2.reshape(1, _D))


def kernel(edge_index, node_feats, edge_feats, eps,
           W1, b1, gamma1, beta1, W2, b2, gamma2, beta2):
    src = edge_index[0]
    dst = edge_index[1]
    psum, pcnt = _sc_scatter(src, dst, node_feats, edge_feats)
    return _tc_mlp(node_feats, psum, pcnt, eps,
                   W1, b1, gamma1, beta1, W2, b2, gamma2, beta2)


# speedup vs baseline: 4.9502x; 1.5048x over previous
import functools

import jax
import jax.numpy as jnp
from jax import lax
from jax.experimental import pallas as pl
from jax.experimental.pallas import tpu as pltpu
from jax.experimental.pallas import tpu_sc as plsc

_N = 10000
_E = 320000
_D = 128
_BN_EPS = 1e-5

_NC = 2
_NS = 16
_NW = _NC * _NS
_EPW = _E // _NW
_C = 80
_NCH = _EPW // _C
_NP = 10240
_RPS = _NP // _NS


def _sc_body(src_h, dst_h, node_h, edge_h, zf_h,
             psum_h,
             srcv0, srcv1, dstv0, dstv1, rows0, rows1, erows0, erows1,
             acc, semi0, semi1, semg0, semg1):
    c = lax.axis_index("c")
    s = lax.axis_index("s")
    wid = s * _NC + c
    r0 = s * _RPS
    nb = _RPS // _C
    srcv = (srcv0, srcv1)
    dstv = (dstv0, dstv1)
    rows = (rows0, rows1)
    erows = (erows0, erows1)
    semi = (semi0, semi1)
    semg = (semg0, semg1)

    def start_loads(b, q):
        base = wid * _EPW + q * _C
        pltpu.async_copy(src_h.at[pl.ds(base, _C)], srcv[b], semi[b])
        pltpu.async_copy(dst_h.at[pl.ds(base, _C)], dstv[b], semi[b])
        pltpu.async_copy(edge_h.at[pl.ds(base, _C)], erows[b], semi[b])

    def wait_loads(b):
        pltpu.make_async_copy(src_h.at[pl.ds(0, _C)], srcv[b], semi[b]).wait()
        pltpu.make_async_copy(dst_h.at[pl.ds(0, _C)], dstv[b], semi[b]).wait()
        pltpu.make_async_copy(edge_h.at[pl.ds(0, _C)], erows[b], semi[b]).wait()

    def wait_gather(b):
        pltpu.make_async_copy(node_h.at[pl.ds(0, _C)], rows[b], semg[b]).wait()

    def scatter(b):
        pltpu.sync_copy(rows[b], acc.at[dstv[b]], add=True)
        pltpu.sync_copy(erows[b], acc.at[dstv[b]], add=True)

    pltpu.sync_copy(zf_h, rows0)
    for i in range(nb):
        pltpu.sync_copy(rows0, acc.at[pl.ds(r0 + i * _C, _C)])
    plsc.subcore_barrier()

    start_loads(0, 0)

    def chunk_pair(j, carry):
        for b in range(2):
            q = 2 * j + b
            wait_loads(b)
            pltpu.async_copy(node_h.at[srcv[b]], rows[b], semg[b])

            @pl.when(q > 0)
            def _():
                wait_gather(1 - b)
                scatter(1 - b)
            start_loads(1 - b, q + 1)
        return carry

    lax.fori_loop(0, (_NCH - 1) // 2, chunk_pair, 0)

    wait_loads(0)
    pltpu.async_copy(node_h.at[srcv[0]], rows[0], semg[0])
    wait_gather(1)
    scatter(1)
    wait_gather(0)
    scatter(0)

    plsc.subcore_barrier()
    for i in range(nb):
        pltpu.sync_copy(acc.at[pl.ds(r0 + i * _C, _C)], rows0)
        pltpu.sync_copy(rows0, psum_h.at[c, pl.ds(r0 + i * _C, _C)])


def _sc_count_body(dst_h, ones_h, zf_h, pcnt_h,
                   dstv, onesb, rows, acc, sem):
    c = lax.axis_index("c")
    s = lax.axis_index("s")
    wid = s * _NC + c
    r0 = s * _RPS
    nb = _RPS // _C

    pltpu.sync_copy(zf_h, rows)
    for i in range(nb):
        pltpu.sync_copy(rows, acc.at[pl.ds(r0 + i * _C, _C)])
    pltpu.sync_copy(ones_h, onesb)
    plsc.subcore_barrier()

    def chunk(j, carry):
        base = wid * _EPW + j * _C
        pltpu.sync_copy(dst_h.at[pl.ds(base, _C)], dstv)
        pltpu.sync_copy(onesb, acc.at[dstv], add=True)
        return carry

    lax.fori_loop(0, _NCH, chunk, 0)

    plsc.subcore_barrier()
    for i in range(nb):
        pltpu.sync_copy(acc.at[pl.ds(r0 + i * _C, _C)], rows)
        pltpu.sync_copy(rows, pcnt_h.at[c, pl.ds(r0 + i * _C, _C)])


def _sc_scatter(src, dst, node_feats, edge_feats):
    zf = jnp.zeros((_C, _D), jnp.float32)
    ones_blk = jnp.ones((_C, _D), jnp.float32)
    mesh = plsc.VectorSubcoreMesh(core_axis_name="c", subcore_axis_name="s",
                                  num_cores=_NC, num_subcores=_NS)
    sum_call = pl.kernel(
        _sc_body,
        out_type=jax.ShapeDtypeStruct((_NC, _NP, _D), jnp.float32),
        mesh=mesh,
        scratch_types=[
            pltpu.VMEM((_C,), jnp.int32),
            pltpu.VMEM((_C,), jnp.int32),
            pltpu.VMEM((_C,), jnp.int32),
            pltpu.VMEM((_C,), jnp.int32),
            pltpu.VMEM((_C, _D), jnp.float32),
            pltpu.VMEM((_C, _D), jnp.float32),
            pltpu.VMEM((_C, _D), jnp.float32),
            pltpu.VMEM((_C, _D), jnp.float32),
            pltpu.VMEM_SHARED((_NP, _D), jnp.float32),
            pltpu.SemaphoreType.DMA,
            pltpu.SemaphoreType.DMA,
            pltpu.SemaphoreType.DMA,
            pltpu.SemaphoreType.DMA,
        ],
    )
    cnt_call = pl.kernel(
        _sc_count_body,
        out_type=jax.ShapeDtypeStruct((_NC, _NP, _D), jnp.float32),
        mesh=mesh,
        scratch_types=[
            pltpu.VMEM((_C,), jnp.int32),
            pltpu.VMEM((_C, _D), jnp.float32),
            pltpu.VMEM((_C, _D), jnp.float32),
            pltpu.VMEM_SHARED((_NP, _D), jnp.float32),
            pltpu.SemaphoreType.DMA,
        ],
    )
    psum = sum_call(src, dst, node_feats, edge_feats, zf)
    pcnt = cnt_call(dst, ones_blk, zf)
    return psum, pcnt


def _tc_body(x_r, ps_r, pc_r, eps_r, w1_r, b1_r, g1_r, be1_r,
             w2_r, b2_r, g2_r, be2_r, o_r):
    summed = ps_r[0, :_N] + ps_r[1, :_N]
    cnt = pc_r[0, :_N, 0:1] + pc_r[1, :_N, 0:1]
    hv = summed / jnp.maximum(cnt, 1.0)
    h = (1.0 + eps_r[0, 0]) * x_r[...] + hv
    y = lax.dot_general(h, w1_r[...], (((1,), (1,)), ((), ())),
                        preferred_element_type=jnp.float32,
                        precision=lax.Precision.HIGHEST)
    y = y + b1_r[...]
    mu = jnp.mean(y, axis=0, keepdims=True)
    d = y - mu
    var = jnp.mean(d * d, axis=0, keepdims=True)
    y = g1_r[...] * (d * lax.rsqrt(var + _BN_EPS)) + be1_r[...]
    y = jnp.maximum(y, 0.0)
    z = lax.dot_general(y, w2_r[...], (((1,), (1,)), ((), ())),
                        preferred_element_type=jnp.float32,
                        precision=lax.Precision.HIGHEST)
    z = z + b2_r[...]
    mu2 = jnp.mean(z, axis=0, keepdims=True)
    d2 = z - mu2
    var2 = jnp.mean(d2 * d2, axis=0, keepdims=True)
    z = g2_r[...] * (d2 * lax.rsqrt(var2 + _BN_EPS)) + be2_r[...]
    o_r[...] = jnp.maximum(z, 0.0)


def _tc_mlp(node_feats, psum, pcnt, eps, W1, b1, g1, be1, W2, b2, g2, be2):
    return pl.pallas_call(
        _tc_body,
        out_shape=jax.ShapeDtypeStruct((_N, _D), jnp.float32),
        compiler_params=pltpu.CompilerParams(
            vmem_limit_bytes=128 * 1024 * 1024),
    )(node_feats, psum, pcnt,
      eps.reshape(1, 1),
      W1, b1.reshape(1, 2 * _D), g1.reshape(1, 2 * _D), be1.reshape(1, 2 * _D),
      W2, b2.reshape(1, _D), g2.reshape(1, _D), be2.reshape(1, _D))


def kernel(edge_index, node_feats, edge_feats, eps,
           W1, b1, gamma1, beta1, W2, b2, gamma2, beta2):
    src = edge_index[0]
    dst = edge_index[1]
    psum, pcnt = _sc_scatter(src, dst, node_feats, edge_feats)
    return _tc_mlp(node_feats, psum, pcnt, eps,
                   W1, b1, gamma1, beta1, W2, b2, gamma2, beta2)


# speedup vs baseline: 5.5415x; 1.1194x over previous
import jax
import jax.numpy as jnp
from jax import lax
from jax.experimental import pallas as pl
from jax.experimental.pallas import tpu as pltpu
from jax.experimental.pallas import tpu_sc as plsc

_N = 10000
_E = 320000
_D = 128
_BN_EPS = 1e-5

_NC = 2
_NS = 16
_NW = _NC * _NS
_EPW = _E // _NW
_C = 80
_NCH = _EPW // _C
_NP = 10240
_RPS = _NP // _NS


def _sc_body(src_h, dst_h, node_h, edge_h, zf_h,
             psum_h,
             srcv0, srcv1, dstv0, dstv1, rows0, rows1, erows0, erows1,
             acc, semi0, semi1, semg0, semg1):
    c = lax.axis_index("c")
    s = lax.axis_index("s")
    wid = s * _NC + c
    r0 = s * _RPS
    nb = _RPS // _C
    srcv = (srcv0, srcv1)
    dstv = (dstv0, dstv1)
    rows = (rows0, rows1)
    erows = (erows0, erows1)
    semi = (semi0, semi1)
    semg = (semg0, semg1)

    def start_loads(b, q):
        base = wid * _EPW + q * _C
        pltpu.async_copy(src_h.at[pl.ds(base, _C)], srcv[b], semi[b])
        pltpu.async_copy(dst_h.at[pl.ds(base, _C)], dstv[b], semi[b])
        pltpu.async_copy(edge_h.at[pl.ds(base, _C)], erows[b], semi[b])

    def wait_loads(b):
        pltpu.make_async_copy(src_h.at[pl.ds(0, _C)], srcv[b], semi[b]).wait()
        pltpu.make_async_copy(dst_h.at[pl.ds(0, _C)], dstv[b], semi[b]).wait()
        pltpu.make_async_copy(edge_h.at[pl.ds(0, _C)], erows[b], semi[b]).wait()

    def wait_gather(b):
        pltpu.make_async_copy(node_h.at[pl.ds(0, _C)], rows[b], semg[b]).wait()

    def scatter(b):
        pltpu.sync_copy(rows[b], acc.at[dstv[b]], add=True)
        pltpu.sync_copy(erows[b], acc.at[dstv[b]], add=True)

    pltpu.sync_copy(zf_h, rows0)
    for i in range(nb):
        pltpu.sync_copy(rows0, acc.at[pl.ds(r0 + i * _C, _C)])
    plsc.subcore_barrier()

    start_loads(0, 0)

    def chunk_pair(j, carry):
        for b in range(2):
            q = 2 * j + b
            wait_loads(b)
            pltpu.async_copy(node_h.at[srcv[b]], rows[b], semg[b])

            @pl.when(q > 0)
            def _():
                wait_gather(1 - b)
                scatter(1 - b)
            start_loads(1 - b, q + 1)
        return carry

    lax.fori_loop(0, (_NCH - 1) // 2, chunk_pair, 0)

    wait_loads(0)
    pltpu.async_copy(node_h.at[srcv[0]], rows[0], semg[0])
    wait_gather(1)
    scatter(1)
    wait_gather(0)
    scatter(0)

    plsc.subcore_barrier()
    for i in range(nb):
        pltpu.sync_copy(acc.at[pl.ds(r0 + i * _C, _C)], rows0)
        pltpu.sync_copy(rows0, psum_h.at[c, pl.ds(r0 + i * _C, _C)])


def _sc_count_body(dst_h, ones_h, zf_h, pcnt_h,
                   dstv0, dstv1, onesb, rows, acc, semi0, semi1):
    c = lax.axis_index("c")
    s = lax.axis_index("s")
    wid = s * _NC + c
    r0 = s * _RPS
    nb = _RPS // _C
    dstv = (dstv0, dstv1)
    semi = (semi0, semi1)

    def wait_load(b):
        pltpu.make_async_copy(dst_h.at[pl.ds(0, _C)], dstv[b], semi[b]).wait()

    pltpu.sync_copy(zf_h, rows)
    for i in range(nb):
        pltpu.sync_copy(rows, acc.at[pl.ds(r0 + i * _C, _C)])
    pltpu.sync_copy(ones_h, onesb)
    plsc.subcore_barrier()

    pltpu.async_copy(dst_h.at[pl.ds(wid * _EPW, _C)], dstv0, semi0)

    def chunk_pair(j, carry):
        for b in range(2):
            q = 2 * j + b
            wait_load(b)
            base = wid * _EPW + (q + 1) * _C
            pltpu.async_copy(dst_h.at[pl.ds(base, _C)], dstv[1 - b],
                             semi[1 - b])
            pltpu.sync_copy(onesb, acc.at[dstv[b]], add=True)
        return carry

    lax.fori_loop(0, (_NCH - 1) // 2, chunk_pair, 0)

    wait_load(0)
    pltpu.sync_copy(onesb, acc.at[dstv0], add=True)

    plsc.subcore_barrier()
    for i in range(nb):
        pltpu.sync_copy(acc.at[pl.ds(r0 + i * _C, _C)], rows)
        pltpu.sync_copy(rows, pcnt_h.at[c, pl.ds(r0 + i * _C, _C)])


def _sc_scatter(src, dst, node_feats, edge_feats):
    zf = jnp.zeros((_C, _D), jnp.float32)
    ones_blk = jnp.ones((_C, _D), jnp.float32)
    mesh = plsc.VectorSubcoreMesh(core_axis_name="c", subcore_axis_name="s",
                                  num_cores=_NC, num_subcores=_NS)
    sum_call = pl.kernel(
        _sc_body,
        out_type=jax.ShapeDtypeStruct((_NC, _NP, _D), jnp.float32),
        mesh=mesh,
        scratch_types=(
            [pltpu.VMEM((_C,), jnp.int32)] * 4 +
            [pltpu.VMEM((_C, _D), jnp.float32)] * 4 +
            [pltpu.VMEM_SHARED((_NP, _D), jnp.float32)] +
            [pltpu.SemaphoreType.DMA] * 4
        ),
    )
    cnt_call = pl.kernel(
        _sc_count_body,
        out_type=jax.ShapeDtypeStruct((_NC, _NP, _D), jnp.float32),
        mesh=mesh,
        scratch_types=(
            [pltpu.VMEM((_C,), jnp.int32)] * 2 +
            [pltpu.VMEM((_C, _D), jnp.float32)] * 2 +
            [pltpu.VMEM_SHARED((_NP, _D), jnp.float32)] +
            [pltpu.SemaphoreType.DMA] * 2
        ),
    )
    psum = sum_call(src, dst, node_feats, edge_feats, zf)
    pcnt = cnt_call(dst, ones_blk, zf)
    return psum, pcnt


def _tc_body(x_r, ps_r, pc_r, eps_r, w1_r, b1_r, g1_r, be1_r,
             w2_r, b2_r, g2_r, be2_r, o_r):
    summed = ps_r[0, :_N] + ps_r[1, :_N]
    cnt = pc_r[0, :_N, 0:1] + pc_r[1, :_N, 0:1]
    hv = summed / jnp.maximum(cnt, 1.0)
    h = (1.0 + eps_r[0, 0]) * x_r[...] + hv
    y = lax.dot_general(h, w1_r[...], (((1,), (1,)), ((), ())),
                        preferred_element_type=jnp.float32,
                        precision=lax.Precision.HIGHEST)
    y = y + b1_r[...]
    mu = jnp.mean(y, axis=0, keepdims=True)
    d = y - mu
    var = jnp.mean(d * d, axis=0, keepdims=True)
    y = g1_r[...] * (d * lax.rsqrt(var + _BN_EPS)) + be1_r[...]
    y = jnp.maximum(y, 0.0)
    z = lax.dot_general(y, w2_r[...], (((1,), (1,)), ((), ())),
                        preferred_element_type=jnp.float32,
                        precision=lax.Precision.HIGHEST)
    z = z + b2_r[...]
    mu2 = jnp.mean(z, axis=0, keepdims=True)
    d2 = z - mu2
    var2 = jnp.mean(d2 * d2, axis=0, keepdims=True)
    z = g2_r[...] * (d2 * lax.rsqrt(var2 + _BN_EPS)) + be2_r[...]
    o_r[...] = jnp.maximum(z, 0.0)


def _tc_mlp(node_feats, psum, pcnt, eps, W1, b1, g1, be1, W2, b2, g2, be2):
    return pl.pallas_call(
        _tc_body,
        out_shape=jax.ShapeDtypeStruct((_N, _D), jnp.float32),
        compiler_params=pltpu.CompilerParams(
            vmem_limit_bytes=128 * 1024 * 1024),
    )(node_feats, psum, pcnt,
      eps.reshape(1, 1),
      W1, b1.reshape(1, 2 * _D), g1.reshape(1, 2 * _D), be1.reshape(1, 2 * _D),
      W2, b2.reshape(1, _D), g2.reshape(1, _D), be2.reshape(1, _D))


def kernel(edge_index, node_feats, edge_feats, eps,
           W1, b1, gamma1, beta1, W2, b2, gamma2, beta2):
    src = edge_index[0]
    dst = edge_index[1]
    psum, pcnt = _sc_scatter(src, dst, node_feats, edge_feats)
    return _tc_mlp(node_feats, psum, pcnt, eps,
                   W1, b1, gamma1, beta1, W2, b2, gamma2, beta2)


# speedup vs baseline: 5.5535x; 1.0022x over previous
import jax
import jax.numpy as jnp
from jax import lax
from jax.experimental import pallas as pl
from jax.experimental.pallas import tpu as pltpu
from jax.experimental.pallas import tpu_sc as plsc

_N = 10000
_E = 320000
_D = 128
_BN_EPS = 1e-5

_NC = 2
_NS = 16
_NW = _NC * _NS
_EPW = _E // _NW
_C = 80
_NCH = _EPW // _C
_NP = 10240
_RPS = _NP // _NS


def _sc_body(src_h, dst_h, node_h, edge_h, zf_h,
             psum_h,
             srcv0, srcv1, dstv0, dstv1, rows0, rows1, erows0, erows1,
             acc, semi0, semi1, semg0, semg1):
    c = lax.axis_index("c")
    s = lax.axis_index("s")
    wid = s * _NC + c
    r0 = s * _RPS
    nb = _RPS // _C
    srcv = (srcv0, srcv1)
    dstv = (dstv0, dstv1)
    rows = (rows0, rows1)
    erows = (erows0, erows1)
    semi = (semi0, semi1)
    semg = (semg0, semg1)

    def start_loads(b, q):
        base = wid * _EPW + q * _C
        pltpu.async_copy(src_h.at[pl.ds(base, _C)], srcv[b], semi[b])
        pltpu.async_copy(dst_h.at[pl.ds(base, _C)], dstv[b], semi[b])
        pltpu.async_copy(edge_h.at[pl.ds(base, _C)], erows[b], semi[b])

    def wait_loads(b):
        pltpu.make_async_copy(src_h.at[pl.ds(0, _C)], srcv[b], semi[b]).wait()
        pltpu.make_async_copy(dst_h.at[pl.ds(0, _C)], dstv[b], semi[b]).wait()
        pltpu.make_async_copy(edge_h.at[pl.ds(0, _C)], erows[b], semi[b]).wait()

    def wait_gather(b):
        pltpu.make_async_copy(node_h.at[pl.ds(0, _C)], rows[b], semg[b]).wait()

    def scatter(b):
        pltpu.sync_copy(erows[b], acc.at[dstv[b]], add=True)
        wait_gather(b)
        pltpu.sync_copy(rows[b], acc.at[dstv[b]], add=True)

    pltpu.sync_copy(zf_h, rows0)
    for i in range(nb):
        pltpu.sync_copy(rows0, acc.at[pl.ds(r0 + i * _C, _C)])
    plsc.subcore_barrier()

    start_loads(0, 0)

    def chunk_pair(j, carry):
        for b in range(2):
            q = 2 * j + b
            wait_loads(b)
            pltpu.async_copy(node_h.at[srcv[b]], rows[b], semg[b])

            @pl.when(q > 0)
            def _():
                scatter(1 - b)
            start_loads(1 - b, q + 1)
        return carry

    lax.fori_loop(0, (_NCH - 1) // 2, chunk_pair, 0)

    wait_loads(0)
    pltpu.async_copy(node_h.at[srcv[0]], rows[0], semg[0])
    scatter(1)
    scatter(0)

    plsc.subcore_barrier()
    for i in range(nb):
        pltpu.sync_copy(acc.at[pl.ds(r0 + i * _C, _C)], rows0)
        pltpu.sync_copy(rows0, psum_h.at[c, pl.ds(r0 + i * _C, _C)])


def _sc_count_body(dst_h, ones_h, zf_h, pcnt_h,
                   dstv0, dstv1, onesb, rows, acc, semi0, semi1):
    c = lax.axis_index("c")
    s = lax.axis_index("s")
    wid = s * _NC + c
    r0 = s * _RPS
    nb = _RPS // _C
    dstv = (dstv0, dstv1)
    semi = (semi0, semi1)

    def wait_load(b):
        pltpu.make_async_copy(dst_h.at[pl.ds(0, _C)], dstv[b], semi[b]).wait()

    pltpu.sync_copy(zf_h, rows)
    for i in range(nb):
        pltpu.sync_copy(rows, acc.at[pl.ds(r0 + i * _C, _C)])
    pltpu.sync_copy(ones_h, onesb)
    plsc.subcore_barrier()

    pltpu.async_copy(dst_h.at[pl.ds(wid * _EPW, _C)], dstv0, semi0)

    def chunk_pair(j, carry):
        for b in range(2):
            q = 2 * j + b
            wait_load(b)
            base = wid * _EPW + (q + 1) * _C
            pltpu.async_copy(dst_h.at[pl.ds(base, _C)], dstv[1 - b],
                             semi[1 - b])
            pltpu.sync_copy(onesb, acc.at[dstv[b]], add=True)
        return carry

    lax.fori_loop(0, (_NCH - 1) // 2, chunk_pair, 0)

    wait_load(0)
    pltpu.sync_copy(onesb, acc.at[dstv0], add=True)

    plsc.subcore_barrier()
    for i in range(nb):
        pltpu.sync_copy(acc.at[pl.ds(r0 + i * _C, _C)], rows)
        pltpu.sync_copy(rows, pcnt_h.at[c, pl.ds(r0 + i * _C, _C)])


def _sc_scatter(src, dst, node_feats, edge_feats):
    zf = jnp.zeros((_C, _D), jnp.float32)
    ones_blk = jnp.ones((_C, _D), jnp.float32)
    mesh = plsc.VectorSubcoreMesh(core_axis_name="c", subcore_axis_name="s",
                                  num_cores=_NC, num_subcores=_NS)
    sum_call = pl.kernel(
        _sc_body,
        out_type=jax.ShapeDtypeStruct((_NC, _NP, _D), jnp.float32),
        mesh=mesh,
        scratch_types=(
            [pltpu.VMEM((_C,), jnp.int32)] * 4 +
            [pltpu.VMEM((_C, _D), jnp.float32)] * 4 +
            [pltpu.VMEM_SHARED((_NP, _D), jnp.float32)] +
            [pltpu.SemaphoreType.DMA] * 4
        ),
    )
    cnt_call = pl.kernel(
        _sc_count_body,
        out_type=jax.ShapeDtypeStruct((_NC, _NP, _D), jnp.float32),
        mesh=mesh,
        scratch_types=(
            [pltpu.VMEM((_C,), jnp.int32)] * 2 +
            [pltpu.VMEM((_C, _D), jnp.float32)] * 2 +
            [pltpu.VMEM_SHARED((_NP, _D), jnp.float32)] +
            [pltpu.SemaphoreType.DMA] * 2
        ),
    )
    psum = sum_call(src, dst, node_feats, edge_feats, zf)
    pcnt = cnt_call(dst, ones_blk, zf)
    return psum, pcnt


def _tc_body(x_r, ps_r, pc_r, eps_r, w1_r, b1_r, g1_r, be1_r,
             w2_r, b2_r, g2_r, be2_r, o_r):
    summed = ps_r[0, :_N] + ps_r[1, :_N]
    cnt = pc_r[0, :_N, 0:1] + pc_r[1, :_N, 0:1]
    hv = summed / jnp.maximum(cnt, 1.0)
    h = (1.0 + eps_r[0, 0]) * x_r[...] + hv
    y = lax.dot_general(h, w1_r[...], (((1,), (1,)), ((), ())),
                        preferred_element_type=jnp.float32,
                        precision=lax.Precision.HIGHEST)
    y = y + b1_r[...]
    mu = jnp.mean(y, axis=0, keepdims=True)
    d = y - mu
    var = jnp.mean(d * d, axis=0, keepdims=True)
    y = g1_r[...] * (d * lax.rsqrt(var + _BN_EPS)) + be1_r[...]
    y = jnp.maximum(y, 0.0)
    z = lax.dot_general(y, w2_r[...], (((1,), (1,)), ((), ())),
                        preferred_element_type=jnp.float32,
                        precision=lax.Precision.HIGHEST)
    z = z + b2_r[...]
    mu2 = jnp.mean(z, axis=0, keepdims=True)
    d2 = z - mu2
    var2 = jnp.mean(d2 * d2, axis=0, keepdims=True)
    z = g2_r[...] * (d2 * lax.rsqrt(var2 + _BN_EPS)) + be2_r[...]
    o_r[...] = jnp.maximum(z, 0.0)


def _tc_mlp(node_feats, psum, pcnt, eps, W1, b1, g1, be1, W2, b2, g2, be2):
    return pl.pallas_call(
        _tc_body,
        out_shape=jax.ShapeDtypeStruct((_N, _D), jnp.float32),
        compiler_params=pltpu.CompilerParams(
            vmem_limit_bytes=128 * 1024 * 1024),
    )(node_feats, psum, pcnt,
      eps.reshape(1, 1),
      W1, b1.reshape(1, 2 * _D), g1.reshape(1, 2 * _D), be1.reshape(1, 2 * _D),
      W2, b2.reshape(1, _D), g2.reshape(1, _D), be2.reshape(1, _D))


def kernel(edge_index, node_feats, edge_feats, eps,
           W1, b1, gamma1, beta1, W2, b2, gamma2, beta2):
    src = edge_index[0]
    dst = edge_index[1]
    psum, pcnt = _sc_scatter(src, dst, node_feats, edge_feats)
    return _tc_mlp(node_feats, psum, pcnt, eps,
                   W1, b1, gamma1, beta1, W2, b2, gamma2, beta2)
